# baseline (speedup 1.0000x reference)
import functools
import math

import jax
import jax.numpy as jnp
from jax import lax
from jax.experimental import pallas as pl
from jax.experimental.pallas import tpu as pltpu
from jax.experimental.pallas import tpu_sc as plsc

B = 1024
L = 200
D = 128
SCALE = math.sqrt(float(D))

NC = 2
NS = 16
NW = NC * NS
CHUNK = 128
RPW = (B * L) // NW
CPW = RPW // CHUNK
NBUF = 5
FIRE_AHEAD = 3
LANES = 16
VECS_PER_ROW = D // LANES

_mesh = plsc.VectorSubcoreMesh(core_axis_name="c", subcore_axis_name="s")


@functools.partial(
    pl.kernel,
    out_type=jax.ShapeDtypeStruct((B * L, D), jnp.float32),
    mesh=_mesh,
    scratch_types=[
        pltpu.VMEM((CPW, CHUNK), jnp.int32),
        [pltpu.VMEM((CHUNK, D), jnp.float32) for _ in range(NBUF)],
        pltpu.VMEM((L, D), jnp.float32),
        [pltpu.SemaphoreType.DMA for _ in range(NBUF)],
        [pltpu.SemaphoreType.DMA for _ in range(NBUF)],
    ],
)
def _emb_kernel(x_hbm, w_hbm, pe_hbm, out_hbm, idx_v, rows, pe_v, gsem, wsem):
    wid = lax.axis_index("s") * NC + lax.axis_index("c")
    base = wid * CPW

    pltpu.sync_copy(x_hbm.at[wid], idx_v)
    pltpu.sync_copy(pe_hbm.at[pl.ds(0, L)], pe_v)

    def fire(t, bt):
        pltpu.async_copy(w_hbm.at[idx_v.at[t]], rows[bt], gsem[bt])

    def drain_gather(b):
        pltpu.make_async_copy(w_hbm.at[idx_v.at[0]], rows[b], gsem[b]).wait()

    def drain_wb(b):
        pltpu.make_async_copy(
            rows[b], out_hbm.at[pl.ds(0, CHUNK)], wsem[b]).wait()

    def step(s, b, do_drain_wb, do_fire):
        drain_gather(b)

        pe_base = lax.rem((base + s) * CHUNK, L)

        def row_body(r, carry):
            pe_row = pe_base + r
            pe_row = jnp.where(pe_row >= L, pe_row - L, pe_row)
            for c in range(VECS_PER_ROW):
                sl = pl.ds(c * LANES, LANES)
                rows[b][r, sl] = rows[b][r, sl] * SCALE + pe_v[pe_row, sl]
            return carry

        lax.fori_loop(0, CHUNK, row_body, 0)

        if do_drain_wb:
            drain_wb((b + FIRE_AHEAD) % NBUF)
        if do_fire:
            fire(s + FIRE_AHEAD, (b + FIRE_AHEAD) % NBUF)
        pltpu.async_copy(
            rows[b], out_hbm.at[pl.ds((base + s) * CHUNK, CHUNK)], wsem[b])

    for t in range(FIRE_AHEAD):
        fire(t, t)

    step(0, 0, False, True)
    step(1, 1, False, True)
    step(2, 2, True, True)
    step(3, 3, True, True)
    step(4, 4, True, True)

    def group_body(g, carry):
        for b in range(NBUF):
            step(NBUF * g + b, b, True, True)
        return carry

    lax.fori_loop(1, CPW // NBUF - 1, group_body, 0)

    step(CPW - 5, 0, True, True)
    step(CPW - 4, 1, True, True)
    step(CPW - 3, 2, False, False)
    step(CPW - 2, 3, False, False)
    step(CPW - 1, 4, False, False)

    for b in range(NBUF):
        drain_wb(b)


def kernel(x, W, pe):
    x2 = x.reshape(NW, CPW, CHUNK)
    out = _emb_kernel(x2, W, pe)
    return out.reshape(B, L, D)


# speedup vs baseline: 2.5772x; 2.5772x over previous
import functools
import math

import jax
import jax.numpy as jnp
from jax import lax
from jax.experimental import pallas as pl
from jax.experimental.pallas import tpu as pltpu
from jax.experimental.pallas import tpu_sc as plsc

B = 1024
L = 200
D = 128
SCALE = math.sqrt(float(D))

NC = 2
NS = 16
NW = NC * NS
CHUNK = 80
RPW = (B * L) // NW
CPW = RPW // CHUNK
NBUF = 5
FIRE_AHEAD = 3
GROUPS = CPW // NBUF
LANES = 16
VECS_PER_ROW = D // LANES

_mesh = plsc.VectorSubcoreMesh(core_axis_name="c", subcore_axis_name="s")


@functools.partial(
    pl.kernel,
    out_type=jax.ShapeDtypeStruct((B * L, D), jnp.float32),
    mesh=_mesh,
    scratch_types=[
        pltpu.VMEM((CPW, CHUNK), jnp.int32),
        [pltpu.VMEM((CHUNK, D), jnp.float32) for _ in range(NBUF)],
        pltpu.VMEM((L, D), jnp.float32),
        [pltpu.SemaphoreType.DMA for _ in range(NBUF)],
        [pltpu.SemaphoreType.DMA for _ in range(NBUF)],
    ],
)
def _emb_kernel(x_hbm, w_hbm, pe_hbm, out_hbm, idx_v, rows, pe_v, gsem, wsem):
    wid = lax.axis_index("s") * NC + lax.axis_index("c")
    base = wid * CPW

    pltpu.sync_copy(x_hbm.at[wid], idx_v)
    pltpu.sync_copy(pe_hbm.at[pl.ds(0, L)], pe_v)

    def fire(t, bt):
        pltpu.async_copy(w_hbm.at[idx_v.at[t]], rows[bt], gsem[bt])

    def drain_gather(b):
        pltpu.make_async_copy(w_hbm.at[idx_v.at[0]], rows[b], gsem[b]).wait()

    def drain_wb(b):
        pltpu.make_async_copy(
            rows[b], out_hbm.at[pl.ds(0, CHUNK)], wsem[b]).wait()

    def compute(b, pe_base):
        def make_body(off):
            def row_body(r, carry):
                for c in range(VECS_PER_ROW):
                    sl = pl.ds(c * LANES, LANES)
                    rows[b][r, sl] = rows[b][r, sl] * SCALE + pe_v[off + r, sl]
                return carry
            return row_body

        n1 = min(CHUNK, L - pe_base)
        lax.fori_loop(0, n1, make_body(pe_base), 0)
        if n1 < CHUNK:
            lax.fori_loop(n1, CHUNK, make_body(pe_base - L), 0)

    def step(s, b, do_drain_wb, do_fire):
        drain_gather(b)
        compute(b, (b * CHUNK) % L)
        if do_drain_wb:
            drain_wb((b + FIRE_AHEAD) % NBUF)
        if do_fire:
            fire(s + FIRE_AHEAD, (b + FIRE_AHEAD) % NBUF)
        pltpu.async_copy(
            rows[b], out_hbm.at[pl.ds((base + s) * CHUNK, CHUNK)], wsem[b])

    for t in range(FIRE_AHEAD):
        fire(t, t)

    step(0, 0, False, True)
    step(1, 1, False, True)
    step(2, 2, True, True)
    step(3, 3, True, True)
    step(4, 4, True, True)

    def group_body(g, carry):
        for b in range(NBUF):
            step(NBUF * g + b, b, True, True)
        return carry

    lax.fori_loop(1, GROUPS - 1, group_body, 0)

    step(CPW - 5, 0, True, True)
    step(CPW - 4, 1, True, True)
    step(CPW - 3, 2, False, False)
    step(CPW - 2, 3, False, False)
    step(CPW - 1, 4, False, False)

    for b in range(NBUF):
        drain_wb(b)


def kernel(x, W, pe):
    x2 = x.reshape(NW, CPW, CHUNK)
    out = _emb_kernel(x2, W, pe)
    return out.reshape(B, L, D)
